# baseline (speedup 1.0000x reference)
import jax
import jax.numpy as jnp
from jax.experimental import pallas as pl
from jax.experimental.pallas import tpu as pltpu

_N_MAX_OBJECTS = 1000
_THRESHOLD_SCORE = 0.5
_THRESHOLD_AREA = 0.25

_N = 20000
_NP = 20480
_ROWS = _NP // 128
_RB = 256
_NBUF = 8
_NFULL = _N // _RB
_TAIL = _N - _NFULL * _RB
_HW = 64 * 64


def _front(scores, labels, classes_ref):
    bits = jax.lax.bitcast_convert_type(scores, jnp.int32)
    n_keep = _N_MAX_OBJECTS

    def val_step(_, carry):
        lo, hi = carry
        mid = lo + (hi - lo) // 2
        cnt = jnp.sum((bits >= mid).astype(jnp.int32))
        big = cnt >= n_keep
        return (jnp.where(big, mid, lo), jnp.where(big, hi, mid))

    t, _ = jax.lax.fori_loop(
        0, 31, val_step, (jnp.int32(0), jnp.int32(0x7F800000)))

    greater = bits > t
    eq = bits == t
    need = n_keep - jnp.sum(greater.astype(jnp.int32))

    lin = jax.lax.broadcasted_iota(jnp.int32, (_ROWS, 128), 0) * 128 + \
        jax.lax.broadcasted_iota(jnp.int32, (_ROWS, 128), 1)

    def idx_step(_, carry):
        lo, hi = carry
        mid = lo + (hi - lo) // 2
        cnt = jnp.sum((eq & (lin < mid)).astype(jnp.int32))
        ok = cnt >= need
        return (jnp.where(ok, lo, mid), jnp.where(ok, mid, hi))

    _, c = jax.lax.fori_loop(0, 16, idx_step, (jnp.int32(0), jnp.int32(_NP)))
    c = jnp.where(need > 0, c, 0)

    topk = greater | (eq & (lin < c))

    lm = jnp.zeros(labels.shape, dtype=jnp.bool_)
    for i in range(classes_ref.shape[0]):
        lm = lm | (labels == classes_ref[i])

    return topk & lm & (scores > _THRESHOLD_SCORE)


def _kernel(scores_ref, labels_ref, classes_ref, masks_ref, out_ref,
            *scratch):
    bufs = scratch[:_NBUF]
    front_s, sems = scratch[_NBUF], scratch[_NBUF + 1]

    def start_full(b, s):
        pltpu.make_async_copy(
            masks_ref.at[pl.ds(b * _RB, _RB)], bufs[s], sems.at[s]).start()

    def start_tail(s):
        pltpu.make_async_copy(
            masks_ref.at[pl.ds(_NFULL * _RB, _TAIL)],
            bufs[s].at[pl.ds(0, _TAIL)], sems.at[s]).start()

    for s in range(_NBUF):
        start_full(s, s)

    front_s[...] = _front(
        scores_ref[...], labels_ref[...], classes_ref).astype(jnp.int32)

    thr = jnp.int32(int(_THRESHOLD_AREA * _HW))
    sub = _RB // 128

    def consume(b, s, rows):
        pltpu.make_async_copy(
            masks_ref.at[pl.ds(b * _RB, rows)],
            bufs[s] if rows == _RB else bufs[s].at[pl.ds(0, rows)],
            sems.at[s],
        ).wait()
        x = bufs[s][...].reshape(sub, 128, _HW)
        cnt = jnp.sum(x, axis=2)
        r0 = b * sub
        ok = (cnt > thr) & (front_s[pl.ds(r0, sub)] != 0)
        out_ref[pl.ds(r0, sub)] = ok.astype(jnp.int32)

    nsteps = _NFULL // _NBUF

    def body(step, carry):
        for s in range(_NBUF):
            b = step * _NBUF + s
            nxt = b + _NBUF

            @pl.when(nxt < _NFULL)
            def _(nxt=nxt, s=s):
                start_full(nxt, s)

            @pl.when(nxt == _NFULL)
            def _(s=s):
                start_tail(s)

            consume(b, s, _RB)
        return carry

    jax.lax.fori_loop(0, nsteps, body, 0)

    for s in range(_NFULL - nsteps * _NBUF):
        consume(nsteps * _NBUF + s, s, _RB)

    consume(_NFULL, _NFULL % _NBUF, _TAIL)

    r0 = (_NFULL + 1) * sub
    left = _ROWS - r0
    out_ref[pl.ds(r0, left)] = jnp.zeros((left, 128), jnp.int32)


def kernel(labels, scores, masks, all_classes):
    n = scores.shape[0]
    _, h, w = masks.shape

    pad = _NP - n
    scores2d = jnp.pad(scores, (0, pad), constant_values=-1.0).reshape(
        _ROWS, 128)
    labels2d = jnp.pad(labels, (0, pad), constant_values=-1).reshape(
        _ROWS, 128)
    masks2d = masks.reshape(n, h * w)

    out2d = pl.pallas_call(
        _kernel,
        in_specs=[
            pl.BlockSpec(memory_space=pltpu.VMEM),
            pl.BlockSpec(memory_space=pltpu.VMEM),
            pl.BlockSpec(memory_space=pltpu.SMEM),
            pl.BlockSpec(memory_space=pltpu.MemorySpace.HBM),
        ],
        out_specs=pl.BlockSpec(memory_space=pltpu.VMEM),
        out_shape=jax.ShapeDtypeStruct((_ROWS, 128), jnp.int32),
        scratch_shapes=(
            [pltpu.VMEM((_RB, _HW), jnp.int32) for _ in range(_NBUF)]
            + [pltpu.VMEM((_ROWS, 128), jnp.int32),
               pltpu.SemaphoreType.DMA((_NBUF,))]
        ),
    )(scores2d, labels2d, all_classes, masks2d)

    return out2d.reshape(_NP)[:n].astype(jnp.bool_)
